# Initial kernel scaffold; baseline (speedup 1.0000x reference)
import jax
import jax.numpy as jnp
from jax.experimental import pallas as pl


def kernel(logits, labels):
    raise NotImplementedError("write your pallas kernel here")


# speedup vs baseline: 35.8186x; 35.8186x over previous
import functools

import jax
import jax.numpy as jnp
from jax import lax
from jax.experimental import pallas as pl
from jax.experimental.pallas import tpu as pltpu
from jax.experimental.pallas import tpu_sc as plsc

C = 21
NPIX = 4 * 384 * 384
PPI = 384 * 384
NC, NS, L = 2, 16, 16
NW = NC * NS
PPT = NPIX // NW
CH = 1024
NCHUNK = PPT // CH
K = 1024
ROW = 2 * K + 16
HIST = C * ROW
GROUPS = CH // L


def _sc_body(logits_hbm, labels_hbm, out_hbm, ebuf, labbuf, rcpbuf, hist):
    wid = lax.axis_index("s") * NC + lax.axis_index("c")
    img = wid // (NW // 4)
    base = (wid % (NW // 4)) * PPT

    zeros = jnp.zeros((L,), jnp.float32)
    ones = jnp.ones((L,), jnp.float32)
    neg_ones = jnp.full((L,), -1.0, jnp.float32)
    iota = lax.iota(jnp.int32, L)
    kf = jnp.float32(K)
    kmax = jnp.full((L,), K - 1, jnp.int32)

    def zero_step(i, _):
        hist[pl.ds(i * L, L)] = zeros
        return 0

    lax.fori_loop(0, HIST // L, zero_step, 0)

    def chunk(t, _):
        off = base + t * CH
        pltpu.sync_copy(logits_hbm.at[img, :, pl.ds(off, CH)], ebuf)
        pltpu.sync_copy(labels_hbm.at[pl.ds(wid * PPT + t * CH, CH)], labbuf)

        def pass_a(g, _):
            s = zeros
            for c in range(C):
                e = jnp.exp(ebuf[c, pl.ds(g * L, L)])
                ebuf[c, pl.ds(g * L, L)] = e
                s = s + e
            rcpbuf[pl.ds(g * L, L)] = 1.0 / s
            return 0

        lax.fori_loop(0, GROUPS, pass_a, 0)

        for c in range(C):
            def pass_b(g, acc, c=c):
                p = ebuf[c, pl.ds(g * L, L)] * rcpbuf[pl.ds(g * L, L)]
                bgm = labbuf[pl.ds(g * L, L)] != c
                b = jnp.minimum((p * kf).astype(jnp.int32), kmax)
                idx = b + (c * ROW)
                plsc.addupdate_scatter(hist, [idx], ones, mask=bgm)
                plsc.addupdate_scatter(hist, [idx + K], p, mask=bgm)
                return acc + p

            acc = lax.fori_loop(0, GROUPS, pass_b, zeros)
            base_t = c * ROW + 2 * K
            hist[pl.ds(base_t, L)] = hist[pl.ds(base_t, L)] + acc
        return 0

    lax.fori_loop(0, NCHUNK, chunk, 0)
    pltpu.sync_copy(hist, out_hbm.at[wid])


@functools.partial(jax.jit, static_argnums=())
def _sc_hist(logits3, labels_flat):
    mesh = plsc.VectorSubcoreMesh(core_axis_name="c", subcore_axis_name="s")
    return pl.kernel(
        _sc_body,
        out_type=jax.ShapeDtypeStruct((NW, HIST), jnp.float32),
        mesh=mesh,
        compiler_params=pltpu.CompilerParams(needs_layout_passes=False),
        scratch_types=[
            pltpu.VMEM((C, CH), jnp.float32),
            pltpu.VMEM((CH,), jnp.int32),
            pltpu.VMEM((CH,), jnp.float32),
            pltpu.VMEM((HIST,), jnp.float32),
        ],
    )(logits3, labels_flat)


def _tc_body(tab_ref, out_ref):
    t = jnp.sum(tab_ref[...], axis=0)
    counts = t[:, 0:K]
    sums = t[:, K:2 * K]
    tot = jnp.sum(t[:, 2 * K:], axis=1, keepdims=True)
    bg = jnp.sum(counts, axis=1, keepdims=True)
    fg = jnp.float32(NPIX) - bg
    ii = lax.broadcasted_iota(jnp.int32, (K, K), 0)
    jj = lax.broadcasted_iota(jnp.int32, (K, K), 1)
    m = (ii > jj).astype(jnp.float32)
    r = jax.lax.dot_general(counts, m, (((1,), (0,)), ((), ())),
                            preferred_element_type=jnp.float32)
    d1 = jnp.maximum(fg + r, 1.0)
    d2 = jnp.maximum(fg + r + counts, 1.0)
    t_c = fg * jnp.sum(sums / (d1 * d2), axis=1, keepdims=True)
    s_fg = tot - jnp.sum(sums, axis=1, keepdims=True)
    loss = 1.0 - s_fg / jnp.float32(NPIX) + t_c
    present = jnp.logical_and(fg > 0.5, bg > 0.5).astype(jnp.float32)
    cnt = jnp.sum(present)
    total = jnp.sum(loss * present)
    res = jnp.where(cnt > 0.0, total / cnt, 0.0)
    out_ref[...] = jnp.reshape(res, (1, 1))


def _tc_combine(table):
    return pl.pallas_call(
        _tc_body,
        out_shape=jax.ShapeDtypeStruct((1, 1), jnp.float32),
    )(table)


def kernel(logits, labels):
    logits3 = logits.reshape(4, C, PPI)
    labels_flat = labels.reshape(-1)
    table = _sc_hist(logits3, labels_flat)
    out = _tc_combine(table.reshape(NW, C, ROW))
    return out[0, 0]


# speedup vs baseline: 36.9295x; 1.0310x over previous
import functools

import jax
import jax.numpy as jnp
from jax import lax
from jax.experimental import pallas as pl
from jax.experimental.pallas import tpu as pltpu
from jax.experimental.pallas import tpu_sc as plsc

C = 21
NPIX = 4 * 384 * 384
PPI = 384 * 384
NC, NS, L = 2, 16, 16
NW = NC * NS
PPT = NPIX // NW
CH = 1024
NCHUNK = PPT // CH
K = 1024
ROW = 2 * K + 16
HIST = C * ROW
GROUPS = CH // L


def _sc_body(logits_hbm, labels_hbm, out_hbm, ebuf, labbuf, rcpbuf, hist):
    wid = lax.axis_index("s") * NC + lax.axis_index("c")
    img = wid // (NW // 4)
    base = (wid % (NW // 4)) * PPT

    zeros = jnp.zeros((L,), jnp.float32)
    ones = jnp.ones((L,), jnp.float32)
    neg_ones = jnp.full((L,), -1.0, jnp.float32)
    iota = lax.iota(jnp.int32, L)
    kf = jnp.float32(K)
    kmax = jnp.full((L,), K - 1, jnp.int32)

    def zero_step(i, _):
        hist[pl.ds(i * L, L)] = zeros
        return 0

    lax.fori_loop(0, HIST // L, zero_step, 0)

    def chunk(t, _):
        off = base + t * CH
        pltpu.sync_copy(logits_hbm.at[img, :, pl.ds(off, CH)], ebuf)
        pltpu.sync_copy(labels_hbm.at[pl.ds(wid * PPT + t * CH, CH)], labbuf)

        def pass_a(g, _):
            s = zeros
            for c in range(C):
                e = jnp.exp(ebuf[c, pl.ds(g * L, L)])
                ebuf[c, pl.ds(g * L, L)] = e
                s = s + e
            rcpbuf[pl.ds(g * L, L)] = 1.0 / s
            return 0

        lax.fori_loop(0, GROUPS, pass_a, 0)

        def pass_b(g, _):
            rcp = rcpbuf[pl.ds(g * L, L)]
            lab = labbuf[pl.ds(g * L, L)]
            sfg = zeros
            for c in range(C):
                p = ebuf[c, pl.ds(g * L, L)] * rcp
                bgm = lab != c
                b = jnp.minimum((p * kf).astype(jnp.int32), kmax)
                idx = b + (c * ROW)
                plsc.addupdate_scatter(hist, [idx], ones, mask=bgm)
                plsc.addupdate_scatter(hist, [idx + K], p, mask=bgm)
                sfg = sfg + jnp.where(bgm, zeros, p)
            plsc.addupdate_scatter(hist, [lab * ROW + (2 * K) + iota], sfg)
            return 0

        lax.fori_loop(0, GROUPS, pass_b, 0)
        return 0

    lax.fori_loop(0, NCHUNK, chunk, 0)
    pltpu.sync_copy(hist, out_hbm.at[wid])


@functools.partial(jax.jit, static_argnums=())
def _sc_hist(logits3, labels_flat):
    mesh = plsc.VectorSubcoreMesh(core_axis_name="c", subcore_axis_name="s")
    return pl.kernel(
        _sc_body,
        out_type=jax.ShapeDtypeStruct((NW, HIST), jnp.float32),
        mesh=mesh,
        compiler_params=pltpu.CompilerParams(needs_layout_passes=False),
        scratch_types=[
            pltpu.VMEM((C, CH), jnp.float32),
            pltpu.VMEM((CH,), jnp.int32),
            pltpu.VMEM((CH,), jnp.float32),
            pltpu.VMEM((HIST,), jnp.float32),
        ],
    )(logits3, labels_flat)


def _tc_body(tab_ref, out_ref):
    t = jnp.sum(tab_ref[...], axis=0)
    counts = t[:, 0:K]
    sums = t[:, K:2 * K]
    s_fg = jnp.sum(t[:, 2 * K:], axis=1, keepdims=True)
    bg = jnp.sum(counts, axis=1, keepdims=True)
    fg = jnp.float32(NPIX) - bg
    ii = lax.broadcasted_iota(jnp.int32, (K, K), 0)
    jj = lax.broadcasted_iota(jnp.int32, (K, K), 1)
    m = (ii > jj).astype(jnp.float32)
    r = jax.lax.dot_general(counts, m, (((1,), (0,)), ((), ())),
                            preferred_element_type=jnp.float32)
    d1 = jnp.maximum(fg + r, 1.0)
    d2 = jnp.maximum(fg + r + counts, 1.0)
    t_c = fg * jnp.sum(sums / (d1 * d2), axis=1, keepdims=True)
    loss = 1.0 - s_fg / jnp.float32(NPIX) + t_c
    present = jnp.logical_and(fg > 0.5, bg > 0.5).astype(jnp.float32)
    cnt = jnp.sum(present)
    total = jnp.sum(loss * present)
    res = jnp.where(cnt > 0.0, total / cnt, 0.0)
    out_ref[...] = jnp.reshape(res, (1, 1))


def _tc_combine(table):
    return pl.pallas_call(
        _tc_body,
        out_shape=jax.ShapeDtypeStruct((1, 1), jnp.float32),
    )(table)


def kernel(logits, labels):
    logits3 = logits.reshape(4, C, PPI)
    labels_flat = labels.reshape(-1)
    table = _sc_hist(logits3, labels_flat)
    out = _tc_combine(table.reshape(NW, C, ROW))
    return out[0, 0]


# speedup vs baseline: 75.4223x; 2.0423x over previous
import functools

import jax
import jax.numpy as jnp
from jax import lax
from jax.experimental import pallas as pl
from jax.experimental.pallas import tpu as pltpu
from jax.experimental.pallas import tpu_sc as plsc

C = 21
NPIX = 4 * 384 * 384
PPI = 384 * 384
NC, NS, L = 2, 16, 16
NW = NC * NS
PPT = NPIX // NW
CH = 1024
NCHUNK = PPT // CH
K = 1024
ROW = 2 * K + 16
HIST = C * ROW
GROUPS = CH // L


def _sc_body(logits_hbm, labels_hbm, out_hbm, ebuf, labbuf, hist):
    wid = lax.axis_index("s") * NC + lax.axis_index("c")
    img = wid // (NW // 4)
    base = (wid % (NW // 4)) * PPT

    zeros = jnp.zeros((L,), jnp.float32)
    ones = jnp.ones((L,), jnp.float32)
    neg_ones = jnp.full((L,), -1.0, jnp.float32)
    iota = lax.iota(jnp.int32, L)
    kf = jnp.float32(K)
    kmax = jnp.full((L,), K - 1, jnp.int32)

    def zero_step(i, _):
        hist[pl.ds(i * L, L)] = zeros
        return 0

    lax.fori_loop(0, HIST // L, zero_step, 0)

    def chunk(t, _):
        off = base + t * CH
        pltpu.sync_copy(logits_hbm.at[img, :, pl.ds(off, CH)], ebuf)
        pltpu.sync_copy(labels_hbm.at[pl.ds(wid * PPT + t * CH, CH)], labbuf)

        def group(g, _):
            lab = labbuf[pl.ds(g * L, L)]
            es = []
            s = zeros
            for c in range(C):
                e = jnp.exp(ebuf[c, pl.ds(g * L, L)])
                es.append(e)
                s = s + e
            rcp = 1.0 / s
            sfg = zeros
            for c in range(C):
                p = es[c] * rcp
                bgm = lab != c
                b = jnp.minimum((p * kf).astype(jnp.int32), kmax)
                idx = b + (c * ROW)
                plsc.addupdate_scatter(hist, [idx], ones, mask=bgm)
                plsc.addupdate_scatter(hist, [idx + K], p, mask=bgm)
                sfg = sfg + jnp.where(bgm, zeros, p)
            plsc.addupdate_scatter(hist, [lab * ROW + (2 * K) + iota], sfg)
            return 0

        lax.fori_loop(0, GROUPS, group, 0)
        return 0

    lax.fori_loop(0, NCHUNK, chunk, 0)
    pltpu.sync_copy(hist, out_hbm.at[wid])


@functools.partial(jax.jit, static_argnums=())
def _sc_hist(logits3, labels_flat):
    mesh = plsc.VectorSubcoreMesh(core_axis_name="c", subcore_axis_name="s")
    return pl.kernel(
        _sc_body,
        out_type=jax.ShapeDtypeStruct((NW, HIST), jnp.float32),
        mesh=mesh,
        compiler_params=pltpu.CompilerParams(needs_layout_passes=False),
        scratch_types=[
            pltpu.VMEM((C, CH), jnp.float32),
            pltpu.VMEM((CH,), jnp.int32),
            pltpu.VMEM((HIST,), jnp.float32),
        ],
    )(logits3, labels_flat)


def _tc_body(tab_ref, out_ref):
    t = jnp.sum(tab_ref[...], axis=0)
    counts = t[:, 0:K]
    sums = t[:, K:2 * K]
    s_fg = jnp.sum(t[:, 2 * K:], axis=1, keepdims=True)
    bg = jnp.sum(counts, axis=1, keepdims=True)
    fg = jnp.float32(NPIX) - bg
    ii = lax.broadcasted_iota(jnp.int32, (K, K), 0)
    jj = lax.broadcasted_iota(jnp.int32, (K, K), 1)
    m = (ii > jj).astype(jnp.float32)
    r = jax.lax.dot_general(counts, m, (((1,), (0,)), ((), ())),
                            preferred_element_type=jnp.float32)
    d1 = jnp.maximum(fg + r, 1.0)
    d2 = jnp.maximum(fg + r + counts, 1.0)
    t_c = fg * jnp.sum(sums / (d1 * d2), axis=1, keepdims=True)
    loss = 1.0 - s_fg / jnp.float32(NPIX) + t_c
    present = jnp.logical_and(fg > 0.5, bg > 0.5).astype(jnp.float32)
    cnt = jnp.sum(present)
    total = jnp.sum(loss * present)
    res = jnp.where(cnt > 0.0, total / cnt, 0.0)
    out_ref[...] = jnp.reshape(res, (1, 1))


def _tc_combine(table):
    return pl.pallas_call(
        _tc_body,
        out_shape=jax.ShapeDtypeStruct((1, 1), jnp.float32),
    )(table)


def kernel(logits, labels):
    logits3 = logits.reshape(4, C, PPI)
    labels_flat = labels.reshape(-1)
    table = _sc_hist(logits3, labels_flat)
    out = _tc_combine(table.reshape(NW, C, ROW))
    return out[0, 0]
